# Initial kernel scaffold; baseline (speedup 1.0000x reference)
import functools

import jax
import jax.numpy as jnp
from jax import lax
from jax.experimental import pallas as pl
from jax.experimental.pallas import tpu as pltpu
from jax.experimental.pallas import tpu_sc as plsc

NC = 2
NS = 16
L = 16
NW = NC * NS
CHUNK = 128


def _mm_body(x_ref, w_ref, o_ref):
    o_ref[...] = jnp.dot(x_ref[...], w_ref[...],
                         preferred_element_type=jnp.float32)


def _matmul_tc(x, w, blk=2000):
    n, k = x.shape
    m = w.shape[1]
    return pl.pallas_call(
        _mm_body,
        grid=(n // blk,),
        in_specs=[pl.BlockSpec((blk, k), lambda i: (i, 0)),
                  pl.BlockSpec((k, m), lambda i: (0, 0))],
        out_specs=pl.BlockSpec((blk, m), lambda i: (i, 0)),
        out_shape=jax.ShapeDtypeStruct((n, m), jnp.float32),
    )(x, w)


def _ram_body(a_ref, b_ref, w_ref, o_ref):
    h = jnp.maximum(a_ref[...] + b_ref[...], 0.0)
    o_ref[...] = jnp.dot(h, w_ref[...], preferred_element_type=jnp.float32)


def _relu_add_matmul_tc(a, b, w, blk=2000):
    n, k = a.shape
    m = w.shape[1]
    return pl.pallas_call(
        _ram_body,
        grid=(n // blk,),
        in_specs=[pl.BlockSpec((blk, k), lambda i: (i, 0)),
                  pl.BlockSpec((blk, k), lambda i: (i, 0)),
                  pl.BlockSpec((k, m), lambda i: (0, 0))],
        out_specs=pl.BlockSpec((blk, m), lambda i: (i, 0)),
        out_shape=jax.ShapeDtypeStruct((n, m), jnp.float32),
    )(a, b, w)


def _add_body(a_ref, b_ref, o_ref):
    o_ref[...] = a_ref[...] + b_ref[...]


def _add_tc(a, b, blk=2000):
    n, m = a.shape
    return pl.pallas_call(
        _add_body,
        grid=(n // blk,),
        in_specs=[pl.BlockSpec((blk, m), lambda i: (i, 0)),
                  pl.BlockSpec((blk, m), lambda i: (i, 0))],
        out_specs=pl.BlockSpec((blk, m), lambda i: (i, 0)),
        out_shape=jax.ShapeDtypeStruct((n, m), jnp.float32),
    )(a, b)


@functools.cache
def _make_propagate(n_nodes, d, n_chunks):
    epw = n_chunks * CHUNK
    rows_per_sub = n_nodes // NS
    nd = d // L
    mesh = plsc.VectorSubcoreMesh(core_axis_name="c", subcore_axis_name="s")

    @functools.partial(
        pl.kernel,
        out_type=jax.ShapeDtypeStruct((NC, n_nodes, d), jnp.float32),
        mesh=mesh,
        scratch_types=[
            pltpu.VMEM((epw,), jnp.int32),
            pltpu.VMEM((n_chunks, CHUNK), jnp.int32),
            pltpu.VMEM((epw,), jnp.float32),
            pltpu.VMEM((CHUNK, d), jnp.float32),
            pltpu.VMEM((rows_per_sub, d), jnp.float32),
            pltpu.VMEM_SHARED((n_nodes, d), jnp.float32),
        ],
    )
    def prop(h_hbm, src_hbm, dst_hbm, ew_hbm, out_hbm,
             src_v, dst_v, ew_v, rows_v, buf_v, acc_s):
        cid = lax.axis_index("c")
        sid = lax.axis_index("s")
        wid = cid * NS + sid
        zrow = jnp.zeros((L,), jnp.float32)
        for i in range(rows_per_sub):
            for k in range(nd):
                buf_v[i, k * L:(k + 1) * L] = zrow
        pltpu.sync_copy(buf_v, acc_s.at[pl.ds(sid * rows_per_sub, rows_per_sub)])
        pltpu.sync_copy(src_hbm.at[wid], src_v)
        pltpu.sync_copy(dst_hbm.at[wid], dst_v)
        pltpu.sync_copy(ew_hbm.at[wid], ew_v)
        plsc.subcore_barrier()

        def chunk_body(j, carry):
            pltpu.sync_copy(h_hbm.at[src_v.at[pl.ds(j * CHUNK, CHUNK)]], rows_v)
            for i in range(CHUNK):
                wb = plsc.load_gather(
                    ew_v, [jnp.full((L,), j * CHUNK + i, jnp.int32)])
                for k in range(nd):
                    rows_v[i, k * L:(k + 1) * L] = (
                        rows_v[i, k * L:(k + 1) * L] * wb)
            pltpu.sync_copy(rows_v, acc_s.at[dst_v.at[j]], add=True)
            return carry

        lax.fori_loop(0, n_chunks, chunk_body, 0)
        plsc.subcore_barrier()
        pltpu.sync_copy(acc_s.at[pl.ds(sid * rows_per_sub, rows_per_sub)], buf_v)
        pltpu.sync_copy(buf_v, out_hbm.at[cid, pl.ds(sid * rows_per_sub,
                                                     rows_per_sub)])

    return prop


def kernel(x, edge_index, edge_weight, W1, W2):
    n, _ = x.shape
    hid = W1.shape[1]
    d_out = W2.shape[1]
    e = edge_index.shape[1]

    n_chunks = -(-e // (NW * CHUNK))
    e_pad = NW * n_chunks * CHUNK
    pad = e_pad - e
    src = jnp.concatenate([edge_index[0], jnp.zeros((pad,), jnp.int32)])
    dst = jnp.concatenate([edge_index[1], jnp.zeros((pad,), jnp.int32)])
    ew = jnp.concatenate([edge_weight, jnp.zeros((pad,), jnp.float32)])
    src = src.reshape(NW, n_chunks * CHUNK)
    dst = dst.reshape(NW, n_chunks, CHUNK)
    ew = ew.reshape(NW, n_chunks * CHUNK)

    prop1 = _make_propagate(n, hid, n_chunks)
    prop2 = _make_propagate(n, d_out, n_chunks)

    h0 = _matmul_tc(x, W1)
    p1 = prop1(h0, src, dst, ew)
    g = _relu_add_matmul_tc(p1[0], p1[1], W2)
    p2 = prop2(g, src, dst, ew)
    return _add_tc(p2[0], p2[1])


# speedup vs baseline: 9.3545x; 9.3545x over previous
import functools

import jax
import jax.numpy as jnp
from jax import lax
from jax.experimental import pallas as pl
from jax.experimental.pallas import tpu as pltpu
from jax.experimental.pallas import tpu_sc as plsc

NC = 2
NS = 16
L = 16
NW = NC * NS
CHUNK = 128


def _mm_body(x_ref, w_ref, o_ref):
    o_ref[...] = jnp.dot(x_ref[...], w_ref[...],
                         preferred_element_type=jnp.float32)


def _matmul_tc(x, w, blk=2000):
    n, k = x.shape
    m = w.shape[1]
    return pl.pallas_call(
        _mm_body,
        grid=(n // blk,),
        in_specs=[pl.BlockSpec((blk, k), lambda i: (i, 0)),
                  pl.BlockSpec((k, m), lambda i: (0, 0))],
        out_specs=pl.BlockSpec((blk, m), lambda i: (i, 0)),
        out_shape=jax.ShapeDtypeStruct((n, m), jnp.float32),
    )(x, w)


def _ram_body(a_ref, b_ref, w_ref, o_ref):
    h = jnp.maximum(a_ref[...] + b_ref[...], 0.0)
    o_ref[...] = jnp.dot(h, w_ref[...], preferred_element_type=jnp.float32)


def _relu_add_matmul_tc(a, b, w, blk=2000):
    n, k = a.shape
    m = w.shape[1]
    return pl.pallas_call(
        _ram_body,
        grid=(n // blk,),
        in_specs=[pl.BlockSpec((blk, k), lambda i: (i, 0)),
                  pl.BlockSpec((blk, k), lambda i: (i, 0)),
                  pl.BlockSpec((k, m), lambda i: (0, 0))],
        out_specs=pl.BlockSpec((blk, m), lambda i: (i, 0)),
        out_shape=jax.ShapeDtypeStruct((n, m), jnp.float32),
    )(a, b, w)


def _add_body(a_ref, b_ref, o_ref):
    o_ref[...] = a_ref[...] + b_ref[...]


def _add_tc(a, b, blk=2000):
    n, m = a.shape
    return pl.pallas_call(
        _add_body,
        grid=(n // blk,),
        in_specs=[pl.BlockSpec((blk, m), lambda i: (i, 0)),
                  pl.BlockSpec((blk, m), lambda i: (i, 0))],
        out_specs=pl.BlockSpec((blk, m), lambda i: (i, 0)),
        out_shape=jax.ShapeDtypeStruct((n, m), jnp.float32),
    )(a, b)


@functools.cache
def _make_propagate(n_acc, d, n_chunks):
    epw = n_chunks * CHUNK
    rows_per_sub = n_acc // NS
    nd = d // L
    mesh = plsc.VectorSubcoreMesh(core_axis_name="c", subcore_axis_name="s")

    @functools.partial(
        pl.kernel,
        out_type=jax.ShapeDtypeStruct((NC, n_acc, d), jnp.float32),
        mesh=mesh,
        scratch_types=[
            pltpu.VMEM((epw,), jnp.int32),
            pltpu.VMEM((n_chunks, CHUNK), jnp.int32),
            pltpu.VMEM((epw,), jnp.float32),
            pltpu.VMEM((CHUNK, d), jnp.float32),
            pltpu.VMEM((rows_per_sub, d), jnp.float32),
            pltpu.VMEM_SHARED((n_acc, d), jnp.float32),
        ],
        compiler_params=pltpu.CompilerParams(use_tc_tiling_on_sc=False),
    )
    def prop(h_hbm, src_hbm, dst_hbm, ew_hbm, out_hbm,
             src_v, dst_v, ew_v, rows_v, buf_v, acc_s):
        cid = lax.axis_index("c")
        sid = lax.axis_index("s")
        wid = cid * NS + sid
        zrow = jnp.zeros((L,), jnp.float32)
        for i in range(rows_per_sub):
            for k in range(nd):
                buf_v[i, k * L:(k + 1) * L] = zrow
        pltpu.sync_copy(buf_v, acc_s.at[pl.ds(sid * rows_per_sub, rows_per_sub)])
        pltpu.sync_copy(src_hbm.at[wid], src_v)
        pltpu.sync_copy(dst_hbm.at[wid], dst_v)
        pltpu.sync_copy(ew_hbm.at[wid], ew_v)
        plsc.subcore_barrier()

        def chunk_body(j, carry):
            pltpu.sync_copy(h_hbm.at[src_v.at[pl.ds(j * CHUNK, CHUNK)]], rows_v)
            for g in range(CHUNK // L):
                w16 = ew_v[pl.ds(j * CHUNK + g * L, L)]
                for i in range(L):
                    wb = jnp.full((L,), w16[i])
                    row = g * L + i
                    for k in range(nd):
                        rows_v[row, k * L:(k + 1) * L] = (
                            rows_v[row, k * L:(k + 1) * L] * wb)
            pltpu.sync_copy(rows_v, acc_s.at[dst_v.at[j]], add=True)
            return carry

        lax.fori_loop(0, n_chunks, chunk_body, 0)
        plsc.subcore_barrier()
        pltpu.sync_copy(acc_s.at[pl.ds(sid * rows_per_sub, rows_per_sub)], buf_v)
        pltpu.sync_copy(buf_v, out_hbm.at[cid, pl.ds(sid * rows_per_sub,
                                                     rows_per_sub)])

    return prop


def kernel(x, edge_index, edge_weight, W1, W2):
    n, _ = x.shape
    hid = W1.shape[1]
    d_out = W2.shape[1]
    e = edge_index.shape[1]

    n_acc = -(-n // (NS * 8)) * (NS * 8)
    n_chunks = -(-e // (NW * CHUNK))
    e_pad = NW * n_chunks * CHUNK
    pad = e_pad - e
    src = jnp.concatenate([edge_index[0], jnp.zeros((pad,), jnp.int32)])
    dst = jnp.concatenate([edge_index[1], jnp.zeros((pad,), jnp.int32)])
    ew = jnp.concatenate([edge_weight, jnp.zeros((pad,), jnp.float32)])
    src = src.reshape(NW, n_chunks * CHUNK)
    dst = dst.reshape(NW, n_chunks, CHUNK)
    ew = ew.reshape(NW, n_chunks * CHUNK)

    prop1 = _make_propagate(n_acc, hid, n_chunks)
    prop2 = _make_propagate(n_acc, d_out, n_chunks)

    h0 = _matmul_tc(x, W1)
    p1 = prop1(h0, src, dst, ew)
    g = _relu_add_matmul_tc(p1[0, :n], p1[1, :n], W2)
    p2 = prop2(g, src, dst, ew)
    return _add_tc(p2[0, :n], p2[1, :n])


# speedup vs baseline: 10.4247x; 1.1144x over previous
import functools

import jax
import jax.numpy as jnp
from jax import lax
from jax.experimental import pallas as pl
from jax.experimental.pallas import tpu as pltpu
from jax.experimental.pallas import tpu_sc as plsc

NC = 2
NS = 16
L = 16
NW = NC * NS
CHUNK = 128


def _mm_body(x_ref, w_ref, o_ref):
    o_ref[...] = jnp.dot(x_ref[...], w_ref[...],
                         preferred_element_type=jnp.float32,
                         precision=jax.lax.Precision.HIGHEST)


def _matmul_tc(x, w, blk=2000):
    n, k = x.shape
    m = w.shape[1]
    return pl.pallas_call(
        _mm_body,
        grid=(n // blk,),
        in_specs=[pl.BlockSpec((blk, k), lambda i: (i, 0)),
                  pl.BlockSpec((k, m), lambda i: (0, 0))],
        out_specs=pl.BlockSpec((blk, m), lambda i: (i, 0)),
        out_shape=jax.ShapeDtypeStruct((n, m), jnp.float32),
    )(x, w)


def _relu_add_body(a_ref, b_ref, o_ref):
    o_ref[...] = jnp.maximum(a_ref[...] + b_ref[...], 0.0)


def _relu_add_tc(a, b, blk=2000):
    n, m = a.shape
    return pl.pallas_call(
        _relu_add_body,
        grid=(n // blk,),
        in_specs=[pl.BlockSpec((blk, m), lambda i: (i, 0)),
                  pl.BlockSpec((blk, m), lambda i: (i, 0))],
        out_specs=pl.BlockSpec((blk, m), lambda i: (i, 0)),
        out_shape=jax.ShapeDtypeStruct((n, m), jnp.float32),
    )(a, b)


def _add_matmul_body(a_ref, b_ref, w_ref, o_ref):
    o_ref[...] = jnp.dot(a_ref[...] + b_ref[...], w_ref[...],
                         preferred_element_type=jnp.float32,
                         precision=jax.lax.Precision.HIGHEST)


def _add_matmul_tc(a, b, w, blk=2000):
    n, k = a.shape
    m = w.shape[1]
    return pl.pallas_call(
        _add_matmul_body,
        grid=(n // blk,),
        in_specs=[pl.BlockSpec((blk, k), lambda i: (i, 0)),
                  pl.BlockSpec((blk, k), lambda i: (i, 0)),
                  pl.BlockSpec((k, m), lambda i: (0, 0))],
        out_specs=pl.BlockSpec((blk, m), lambda i: (i, 0)),
        out_shape=jax.ShapeDtypeStruct((n, m), jnp.float32),
    )(a, b, w)


@functools.cache
def _make_propagate(n_acc, d, n_chunks):
    epw = n_chunks * CHUNK
    rows_per_sub = n_acc // NS
    nd = d // L
    mesh = plsc.VectorSubcoreMesh(core_axis_name="c", subcore_axis_name="s")

    @functools.partial(
        pl.kernel,
        out_type=jax.ShapeDtypeStruct((NC, n_acc, d), jnp.float32),
        mesh=mesh,
        scratch_types=[
            pltpu.VMEM((epw,), jnp.int32),
            pltpu.VMEM((n_chunks, CHUNK), jnp.int32),
            pltpu.VMEM((epw,), jnp.float32),
            pltpu.VMEM((CHUNK, d), jnp.float32),
            pltpu.VMEM((rows_per_sub, d), jnp.float32),
            pltpu.VMEM_SHARED((n_acc, d), jnp.float32),
        ],
        compiler_params=pltpu.CompilerParams(use_tc_tiling_on_sc=False),
    )
    def prop(h_hbm, src_hbm, dst_hbm, ew_hbm, out_hbm,
             src_v, dst_v, ew_v, rows_v, buf_v, acc_s):
        cid = lax.axis_index("c")
        sid = lax.axis_index("s")
        wid = cid * NS + sid
        zrow = jnp.zeros((L,), jnp.float32)
        for i in range(rows_per_sub):
            for k in range(nd):
                buf_v[i, k * L:(k + 1) * L] = zrow
        pltpu.sync_copy(buf_v, acc_s.at[pl.ds(sid * rows_per_sub, rows_per_sub)])
        pltpu.sync_copy(src_hbm.at[wid], src_v)
        pltpu.sync_copy(dst_hbm.at[wid], dst_v)
        pltpu.sync_copy(ew_hbm.at[wid], ew_v)
        plsc.subcore_barrier()

        def chunk_body(j, carry):
            pltpu.sync_copy(h_hbm.at[src_v.at[pl.ds(j * CHUNK, CHUNK)]], rows_v)
            for g in range(CHUNK // L):
                w16 = ew_v[pl.ds(j * CHUNK + g * L, L)]
                for i in range(L):
                    wb = jnp.full((L,), w16[i])
                    row = g * L + i
                    for k in range(nd):
                        rows_v[row, k * L:(k + 1) * L] = (
                            rows_v[row, k * L:(k + 1) * L] * wb)
            pltpu.sync_copy(rows_v, acc_s.at[dst_v.at[j]], add=True)
            return carry

        lax.fori_loop(0, n_chunks, chunk_body, 0)
        plsc.subcore_barrier()
        pltpu.sync_copy(acc_s.at[pl.ds(sid * rows_per_sub, rows_per_sub)], buf_v)
        pltpu.sync_copy(buf_v, out_hbm.at[cid, pl.ds(sid * rows_per_sub,
                                                     rows_per_sub)])

    return prop


def kernel(x, edge_index, edge_weight, W1, W2):
    n, _ = x.shape
    hid = W1.shape[1]
    d_out = W2.shape[1]
    e = edge_index.shape[1]

    n_acc = -(-n // (NS * 8)) * (NS * 8)
    n_chunks = -(-e // (NW * CHUNK))
    e_pad = NW * n_chunks * CHUNK
    pad = e_pad - e
    src = jnp.concatenate([edge_index[0], jnp.zeros((pad,), jnp.int32)])
    dst = jnp.concatenate([edge_index[1], jnp.zeros((pad,), jnp.int32)])
    ew = jnp.concatenate([edge_weight, jnp.zeros((pad,), jnp.float32)])
    src = src.reshape(NW, n_chunks * CHUNK)
    dst = dst.reshape(NW, n_chunks, CHUNK)
    ew = ew.reshape(NW, n_chunks * CHUNK)

    prop = _make_propagate(n_acc, hid, n_chunks)

    h0 = _matmul_tc(x, W1)
    p1 = prop(h0, src, dst, ew)
    g = _relu_add_tc(p1[0, :n], p1[1, :n])
    p2 = prop(g, src, dst, ew)
    return _add_matmul_tc(p2[0, :n], p2[1, :n], W2)


# speedup vs baseline: 13.3078x; 1.2766x over previous
import functools

import jax
import jax.numpy as jnp
from jax import lax
from jax.experimental import pallas as pl
from jax.experimental.pallas import tpu as pltpu
from jax.experimental.pallas import tpu_sc as plsc

NC = 2
NS = 16
L = 16
NW = NC * NS
CHUNK = 128


def _mm_body(x_ref, w_ref, o_ref):
    o_ref[...] = jnp.dot(x_ref[...], w_ref[...],
                         preferred_element_type=jnp.float32,
                         precision=jax.lax.Precision.HIGHEST)


def _matmul_tc(x, w, blk=2000):
    n, k = x.shape
    m = w.shape[1]
    return pl.pallas_call(
        _mm_body,
        grid=(n // blk,),
        in_specs=[pl.BlockSpec((blk, k), lambda i: (i, 0)),
                  pl.BlockSpec((k, m), lambda i: (0, 0))],
        out_specs=pl.BlockSpec((blk, m), lambda i: (i, 0)),
        out_shape=jax.ShapeDtypeStruct((n, m), jnp.float32),
    )(x, w)


def _relu_add_body(a_ref, b_ref, o_ref):
    o_ref[...] = jnp.maximum(a_ref[...] + b_ref[...], 0.0)


def _relu_add_tc(a, b, blk=2000):
    n, m = a.shape
    return pl.pallas_call(
        _relu_add_body,
        grid=(n // blk,),
        in_specs=[pl.BlockSpec((blk, m), lambda i: (i, 0)),
                  pl.BlockSpec((blk, m), lambda i: (i, 0))],
        out_specs=pl.BlockSpec((blk, m), lambda i: (i, 0)),
        out_shape=jax.ShapeDtypeStruct((n, m), jnp.float32),
    )(a, b)


def _add_matmul_body(a_ref, b_ref, w_ref, o_ref):
    o_ref[...] = jnp.dot(a_ref[...] + b_ref[...], w_ref[...],
                         preferred_element_type=jnp.float32,
                         precision=jax.lax.Precision.HIGHEST)


def _add_matmul_tc(a, b, w, blk=2000):
    n, k = a.shape
    m = w.shape[1]
    return pl.pallas_call(
        _add_matmul_body,
        grid=(n // blk,),
        in_specs=[pl.BlockSpec((blk, k), lambda i: (i, 0)),
                  pl.BlockSpec((blk, k), lambda i: (i, 0)),
                  pl.BlockSpec((k, m), lambda i: (0, 0))],
        out_specs=pl.BlockSpec((blk, m), lambda i: (i, 0)),
        out_shape=jax.ShapeDtypeStruct((n, m), jnp.float32),
    )(a, b, w)


@functools.cache
def _make_propagate(n_acc, d, n_chunks):
    epw = n_chunks * CHUNK
    rows_per_sub = n_acc // NS
    nd = d // L
    mesh = plsc.VectorSubcoreMesh(core_axis_name="c", subcore_axis_name="s")

    nbuf = 4
    lead = 2
    assert n_chunks % nbuf == 0 and n_chunks >= 2 * nbuf

    @functools.partial(
        pl.kernel,
        out_type=jax.ShapeDtypeStruct((NC, n_acc, d), jnp.float32),
        mesh=mesh,
        scratch_types=[
            pltpu.VMEM((epw,), jnp.int32),
            pltpu.VMEM((n_chunks, CHUNK), jnp.int32),
            pltpu.VMEM((epw,), jnp.float32),
            pltpu.VMEM((nbuf, CHUNK, d), jnp.float32),
            pltpu.VMEM((rows_per_sub, d), jnp.float32),
            pltpu.VMEM_SHARED((n_acc, d), jnp.float32),
            pltpu.SemaphoreType.DMA((nbuf,)),
            pltpu.SemaphoreType.DMA((nbuf,)),
        ],
        compiler_params=pltpu.CompilerParams(use_tc_tiling_on_sc=False),
    )
    def prop(h_hbm, src_hbm, dst_hbm, ew_hbm, out_hbm,
             src_v, dst_v, ew_v, rows_v, buf_v, acc_s, gsem, ssem):
        cid = lax.axis_index("c")
        sid = lax.axis_index("s")
        wid = cid * NS + sid
        zrow = jnp.zeros((L,), jnp.float32)
        for i in range(rows_per_sub):
            for k in range(nd):
                buf_v[i, k * L:(k + 1) * L] = zrow
        pltpu.sync_copy(buf_v, acc_s.at[pl.ds(sid * rows_per_sub, rows_per_sub)])
        pltpu.sync_copy(src_hbm.at[wid], src_v)
        pltpu.sync_copy(dst_hbm.at[wid], dst_v)
        pltpu.sync_copy(ew_hbm.at[wid], ew_v)
        plsc.subcore_barrier()

        def gather_start(c, b):
            pltpu.async_copy(h_hbm.at[src_v.at[pl.ds(c * CHUNK, CHUNK)]],
                             rows_v.at[b], gsem.at[b])

        def gather_wait(c, b):
            pltpu.make_async_copy(
                h_hbm.at[src_v.at[pl.ds(c * CHUNK, CHUNK)]],
                rows_v.at[b], gsem.at[b]).wait()

        def scatter_start(c, b):
            pltpu.async_copy(rows_v.at[b], acc_s.at[dst_v.at[c]],
                             ssem.at[b], add=True)

        def scatter_wait(c, b):
            pltpu.make_async_copy(rows_v.at[b], acc_s.at[dst_v.at[c]],
                                  ssem.at[b]).wait()

        def scale(c, b):
            for g in range(CHUNK // L):
                w16 = ew_v[pl.ds(c * CHUNK + g * L, L)]
                for i in range(L):
                    wb = jnp.full((L,), w16[i])
                    row = g * L + i
                    for k in range(nd):
                        rows_v[b, row, k * L:(k + 1) * L] = (
                            rows_v[b, row, k * L:(k + 1) * L] * wb)

        gather_start(0, 0)
        gather_start(1, 1)
        for b in range(nbuf):
            gather_wait(b, b)
            scale(b, b)
            scatter_start(b, b)
            bn = (b + lead) % nbuf
            if b + lead < nbuf:
                gather_start(b + lead, bn)
            else:
                scatter_wait(b + lead - nbuf, bn)
                gather_start(b + lead, bn)

        def ring_cycle(it, carry):
            j = it * nbuf
            for b in range(nbuf):
                c = j + b
                gather_wait(c, b)
                scale(c, b)
                scatter_start(c, b)
                bn = (b + lead) % nbuf

                @pl.when(c + lead < n_chunks)
                def _():
                    scatter_wait(c + lead - nbuf, bn)
                    gather_start(c + lead, bn)
            return carry

        lax.fori_loop(1, n_chunks // nbuf, ring_cycle, 0)
        for b in range(nbuf):
            scatter_wait(n_chunks - nbuf + b, b)

        plsc.subcore_barrier()
        pltpu.sync_copy(acc_s.at[pl.ds(sid * rows_per_sub, rows_per_sub)], buf_v)
        pltpu.sync_copy(buf_v, out_hbm.at[cid, pl.ds(sid * rows_per_sub,
                                                     rows_per_sub)])

    return prop


def kernel(x, edge_index, edge_weight, W1, W2):
    n, _ = x.shape
    hid = W1.shape[1]
    d_out = W2.shape[1]
    e = edge_index.shape[1]

    n_acc = -(-n // (NS * 8)) * (NS * 8)
    n_chunks = -(-e // (NW * CHUNK))
    n_chunks = max(-(-n_chunks // 4) * 4, 8)
    e_pad = NW * n_chunks * CHUNK
    pad = e_pad - e
    src = jnp.concatenate([edge_index[0], jnp.zeros((pad,), jnp.int32)])
    dst = jnp.concatenate([edge_index[1], jnp.zeros((pad,), jnp.int32)])
    ew = jnp.concatenate([edge_weight, jnp.zeros((pad,), jnp.float32)])
    src = src.reshape(NW, n_chunks * CHUNK)
    dst = dst.reshape(NW, n_chunks, CHUNK)
    ew = ew.reshape(NW, n_chunks * CHUNK)

    prop = _make_propagate(n_acc, hid, n_chunks)

    h0 = _matmul_tc(x, W1)
    p1 = prop(h0, src, dst, ew)
    g = _relu_add_tc(p1[0, :n], p1[1, :n])
    p2 = prop(g, src, dst, ew)
    return _add_matmul_tc(p2[0, :n], p2[1, :n], W2)


# speedup vs baseline: 19.3850x; 1.4567x over previous
import functools

import jax
import jax.numpy as jnp
from jax import lax
from jax.experimental import pallas as pl
from jax.experimental.pallas import tpu as pltpu
from jax.experimental.pallas import tpu_sc as plsc

NC = 2
NS = 16
L = 16
NW = NC * NS
CHUNK = 128


def _mm_body(x_ref, w_ref, o_ref):
    o_ref[...] = jnp.dot(x_ref[...], w_ref[...],
                         preferred_element_type=jnp.float32,
                         precision=jax.lax.Precision.HIGHEST)


def _matmul_tc(x, w, blk=2000):
    n, k = x.shape
    m = w.shape[1]
    return pl.pallas_call(
        _mm_body,
        grid=(n // blk,),
        in_specs=[pl.BlockSpec((blk, k), lambda i: (i, 0)),
                  pl.BlockSpec((k, m), lambda i: (0, 0))],
        out_specs=pl.BlockSpec((blk, m), lambda i: (i, 0)),
        out_shape=jax.ShapeDtypeStruct((n, m), jnp.float32),
    )(x, w)


def _relu_add_body(a_ref, b_ref, o_ref):
    o_ref[...] = jnp.maximum(a_ref[...] + b_ref[...], 0.0)


def _relu_add_tc(a, b, blk=2000):
    n, m = a.shape
    return pl.pallas_call(
        _relu_add_body,
        grid=(n // blk,),
        in_specs=[pl.BlockSpec((blk, m), lambda i: (i, 0)),
                  pl.BlockSpec((blk, m), lambda i: (i, 0))],
        out_specs=pl.BlockSpec((blk, m), lambda i: (i, 0)),
        out_shape=jax.ShapeDtypeStruct((n, m), jnp.float32),
    )(a, b)


def _add_matmul_body(a_ref, b_ref, w_ref, o_ref):
    o_ref[...] = jnp.dot(a_ref[...] + b_ref[...], w_ref[...],
                         preferred_element_type=jnp.float32,
                         precision=jax.lax.Precision.HIGHEST)


def _add_matmul_tc(a, b, w, blk=2000):
    n, k = a.shape
    m = w.shape[1]
    return pl.pallas_call(
        _add_matmul_body,
        grid=(n // blk,),
        in_specs=[pl.BlockSpec((blk, k), lambda i: (i, 0)),
                  pl.BlockSpec((blk, k), lambda i: (i, 0)),
                  pl.BlockSpec((k, m), lambda i: (0, 0))],
        out_specs=pl.BlockSpec((blk, m), lambda i: (i, 0)),
        out_shape=jax.ShapeDtypeStruct((n, m), jnp.float32),
    )(a, b, w)


@functools.cache
def _make_propagate(n_acc, d, n_chunks):
    epw = n_chunks * CHUNK
    rows_per_sub = n_acc // NS
    nd = d // L
    mesh = plsc.VectorSubcoreMesh(core_axis_name="c", subcore_axis_name="s")

    nbuf = 4
    lead = 2
    assert n_chunks % nbuf == 0 and n_chunks >= 2 * nbuf

    @functools.partial(
        pl.kernel,
        out_type=jax.ShapeDtypeStruct((NC, n_acc, d), jnp.float32),
        mesh=mesh,
        scratch_types=[
            pltpu.VMEM((epw,), jnp.int32),
            pltpu.VMEM((n_chunks, CHUNK), jnp.int32),
            pltpu.VMEM((epw,), jnp.float32),
            pltpu.VMEM((nbuf, CHUNK, d), jnp.float32),
            pltpu.VMEM((rows_per_sub, d), jnp.float32),
            pltpu.VMEM_SHARED((n_acc, d), jnp.float32),
            pltpu.VMEM_SHARED((n_acc, d), jnp.float32),
            pltpu.SemaphoreType.DMA((nbuf,)),
            pltpu.SemaphoreType.DMA((nbuf,)),
        ],
        compiler_params=pltpu.CompilerParams(use_tc_tiling_on_sc=False),
    )
    def prop(h_hbm, src_hbm, dst_hbm, ew_hbm, out_hbm,
             src_v, dst_v, ew_v, rows_v, buf_v, acc_s, tab_s, gsem, ssem):
        cid = lax.axis_index("c")
        sid = lax.axis_index("s")
        wid = cid * NS + sid
        n_rows = h_hbm.shape[0]
        full_subs = n_rows // rows_per_sub
        rem = n_rows % rows_per_sub

        @pl.when(sid < full_subs)
        def _():
            pltpu.sync_copy(h_hbm.at[pl.ds(sid * rows_per_sub, rows_per_sub)],
                            buf_v)
            pltpu.sync_copy(buf_v,
                            tab_s.at[pl.ds(sid * rows_per_sub, rows_per_sub)])

        if rem > 0:
            @pl.when(sid == full_subs)
            def _():
                pltpu.sync_copy(h_hbm.at[pl.ds(full_subs * rows_per_sub, rem)],
                                buf_v.at[pl.ds(0, rem)])
                pltpu.sync_copy(buf_v.at[pl.ds(0, rem)],
                                tab_s.at[pl.ds(full_subs * rows_per_sub, rem)])

        zrow = jnp.zeros((L,), jnp.float32)
        for i in range(rows_per_sub):
            for k in range(nd):
                buf_v[i, k * L:(k + 1) * L] = zrow
        pltpu.sync_copy(buf_v, acc_s.at[pl.ds(sid * rows_per_sub, rows_per_sub)])
        pltpu.sync_copy(src_hbm.at[wid], src_v)
        pltpu.sync_copy(dst_hbm.at[wid], dst_v)
        pltpu.sync_copy(ew_hbm.at[wid], ew_v)
        plsc.subcore_barrier()

        def gather_start(c, b):
            pltpu.async_copy(tab_s.at[src_v.at[pl.ds(c * CHUNK, CHUNK)]],
                             rows_v.at[b], gsem.at[b])

        def gather_wait(c, b):
            pltpu.make_async_copy(
                tab_s.at[src_v.at[pl.ds(c * CHUNK, CHUNK)]],
                rows_v.at[b], gsem.at[b]).wait()

        def scatter_start(c, b):
            pltpu.async_copy(rows_v.at[b], acc_s.at[dst_v.at[c]],
                             ssem.at[b], add=True)

        def scatter_wait(c, b):
            pltpu.make_async_copy(rows_v.at[b], acc_s.at[dst_v.at[c]],
                                  ssem.at[b]).wait()

        def scale(c, b):
            for g in range(CHUNK // L):
                w16 = ew_v[pl.ds(c * CHUNK + g * L, L)]
                for i in range(L):
                    wb = jnp.full((L,), w16[i])
                    row = g * L + i
                    for k in range(nd):
                        rows_v[b, row, k * L:(k + 1) * L] = (
                            rows_v[b, row, k * L:(k + 1) * L] * wb)

        gather_start(0, 0)
        gather_start(1, 1)
        for b in range(nbuf):
            gather_wait(b, b)
            scale(b, b)
            scatter_start(b, b)
            bn = (b + lead) % nbuf
            if b + lead < nbuf:
                gather_start(b + lead, bn)
            else:
                scatter_wait(b + lead - nbuf, bn)
                gather_start(b + lead, bn)

        def ring_cycle(it, carry):
            j = it * nbuf
            for b in range(nbuf):
                c = j + b
                gather_wait(c, b)
                scale(c, b)
                scatter_start(c, b)
                bn = (b + lead) % nbuf

                @pl.when(c + lead < n_chunks)
                def _():
                    scatter_wait(c + lead - nbuf, bn)
                    gather_start(c + lead, bn)
            return carry

        lax.fori_loop(1, n_chunks // nbuf, ring_cycle, 0)
        for b in range(nbuf):
            scatter_wait(n_chunks - nbuf + b, b)

        plsc.subcore_barrier()
        pltpu.sync_copy(acc_s.at[pl.ds(sid * rows_per_sub, rows_per_sub)], buf_v)
        pltpu.sync_copy(buf_v, out_hbm.at[cid, pl.ds(sid * rows_per_sub,
                                                     rows_per_sub)])

    return prop


def kernel(x, edge_index, edge_weight, W1, W2):
    n, _ = x.shape
    hid = W1.shape[1]
    d_out = W2.shape[1]
    e = edge_index.shape[1]

    n_acc = -(-n // (NS * 8)) * (NS * 8)
    n_chunks = -(-e // (NW * CHUNK))
    n_chunks = max(-(-n_chunks // 4) * 4, 8)
    e_pad = NW * n_chunks * CHUNK
    pad = e_pad - e
    src = jnp.concatenate([edge_index[0], jnp.zeros((pad,), jnp.int32)])
    dst = jnp.concatenate([edge_index[1], jnp.zeros((pad,), jnp.int32)])
    ew = jnp.concatenate([edge_weight, jnp.zeros((pad,), jnp.float32)])
    src = src.reshape(NW, n_chunks * CHUNK)
    dst = dst.reshape(NW, n_chunks, CHUNK)
    ew = ew.reshape(NW, n_chunks * CHUNK)

    prop = _make_propagate(n_acc, hid, n_chunks)

    h0 = _matmul_tc(x, W1)
    p1 = prop(h0, src, dst, ew)
    g = _relu_add_tc(p1[0, :n], p1[1, :n])
    p2 = prop(g, src, dst, ew)
    return _add_matmul_tc(p2[0, :n], p2[1, :n], W2)


# speedup vs baseline: 23.6195x; 1.2184x over previous
import functools

import jax
import jax.numpy as jnp
from jax import lax
from jax.experimental import pallas as pl
from jax.experimental.pallas import tpu as pltpu
from jax.experimental.pallas import tpu_sc as plsc

NC = 2
NS = 16
L = 16
NW = NC * NS
CHUNK = 128


def _mm_body(x_ref, w_ref, o_ref):
    o_ref[...] = jnp.dot(x_ref[...], w_ref[...],
                         preferred_element_type=jnp.float32,
                         precision=jax.lax.Precision.HIGHEST)


def _matmul_tc(x, w, blk=2000):
    n, k = x.shape
    m = w.shape[1]
    return pl.pallas_call(
        _mm_body,
        grid=(n // blk,),
        in_specs=[pl.BlockSpec((blk, k), lambda i: (i, 0)),
                  pl.BlockSpec((k, m), lambda i: (0, 0))],
        out_specs=pl.BlockSpec((blk, m), lambda i: (i, 0)),
        out_shape=jax.ShapeDtypeStruct((n, m), jnp.float32),
    )(x, w)


def _relu_add_body(a_ref, b_ref, o_ref):
    o_ref[...] = jnp.maximum(a_ref[...] + b_ref[...], 0.0)


def _relu_add_tc(a, b, blk=2000):
    n, m = a.shape
    return pl.pallas_call(
        _relu_add_body,
        grid=(n // blk,),
        in_specs=[pl.BlockSpec((blk, m), lambda i: (i, 0)),
                  pl.BlockSpec((blk, m), lambda i: (i, 0))],
        out_specs=pl.BlockSpec((blk, m), lambda i: (i, 0)),
        out_shape=jax.ShapeDtypeStruct((n, m), jnp.float32),
    )(a, b)


def _add_matmul_body(a_ref, b_ref, w_ref, o_ref):
    o_ref[...] = jnp.dot(a_ref[...] + b_ref[...], w_ref[...],
                         preferred_element_type=jnp.float32,
                         precision=jax.lax.Precision.HIGHEST)


def _add_matmul_tc(a, b, w, blk=2000):
    n, k = a.shape
    m = w.shape[1]
    return pl.pallas_call(
        _add_matmul_body,
        grid=(n // blk,),
        in_specs=[pl.BlockSpec((blk, k), lambda i: (i, 0)),
                  pl.BlockSpec((blk, k), lambda i: (i, 0)),
                  pl.BlockSpec((k, m), lambda i: (0, 0))],
        out_specs=pl.BlockSpec((blk, m), lambda i: (i, 0)),
        out_shape=jax.ShapeDtypeStruct((n, m), jnp.float32),
    )(a, b, w)


@functools.cache
def _make_propagate(n_acc, d, n_in, e, staged):
    epw = e // NW
    assert e % (NW * L) == 0
    n_chunks = -(-epw // CHUNK)
    n_chunks = max(-(-n_chunks // 4) * 4, 8)
    epw_pad = n_chunks * CHUNK
    tail = epw_pad - epw
    assert tail % L == 0
    rows_per_sub = n_acc // NS
    nd = d // L
    mesh = plsc.VectorSubcoreMesh(core_axis_name="c", subcore_axis_name="s")

    nbuf = 4
    lead = 2

    @functools.partial(
        pl.kernel,
        out_type=jax.ShapeDtypeStruct((NC, n_acc, d), jnp.float32),
        mesh=mesh,
        scratch_types=[
            pltpu.VMEM((epw_pad,), jnp.int32),
            pltpu.VMEM((epw_pad,), jnp.int32),
            pltpu.VMEM((epw_pad,), jnp.float32),
            pltpu.VMEM((nbuf, CHUNK, d), jnp.float32),
            pltpu.VMEM((rows_per_sub, d), jnp.float32),
            pltpu.VMEM((rows_per_sub, d), jnp.float32),
            pltpu.VMEM_SHARED((n_acc, d), jnp.float32),
            pltpu.VMEM_SHARED((n_acc, d), jnp.float32),
            pltpu.SemaphoreType.DMA((nbuf,)),
            pltpu.SemaphoreType.DMA((nbuf,)),
        ],
        compiler_params=pltpu.CompilerParams(use_tc_tiling_on_sc=False),
    )
    def prop(h_hbm, ei_hbm, ew_hbm, out_hbm,
             src_v, dst_v, ew_v, rows_v, buf_v, buf2_v, acc_s, tab_s,
             gsem, ssem):
        cid = lax.axis_index("c")
        sid = lax.axis_index("s")
        wid = cid * NS + sid

        if staged:
            pltpu.sync_copy(h_hbm.at[0, pl.ds(sid * rows_per_sub,
                                              rows_per_sub)], buf_v)
            pltpu.sync_copy(h_hbm.at[1, pl.ds(sid * rows_per_sub,
                                              rows_per_sub)], buf2_v)

            def stage_body(i, carry):
                for k in range(nd):
                    a = buf_v[i, k * L:(k + 1) * L]
                    b = buf2_v[i, k * L:(k + 1) * L]
                    buf_v[i, k * L:(k + 1) * L] = jnp.maximum(a + b, 0.0)
                return carry

            lax.fori_loop(0, rows_per_sub, stage_body, 0)
            pltpu.sync_copy(buf_v,
                            tab_s.at[pl.ds(sid * rows_per_sub, rows_per_sub)])
        else:
            n_rows = h_hbm.shape[0]
            full_subs = n_rows // rows_per_sub
            rem = n_rows % rows_per_sub

            @pl.when(sid < full_subs)
            def _():
                pltpu.sync_copy(
                    h_hbm.at[pl.ds(sid * rows_per_sub, rows_per_sub)], buf_v)
                pltpu.sync_copy(
                    buf_v, tab_s.at[pl.ds(sid * rows_per_sub, rows_per_sub)])

            if rem > 0:
                @pl.when(sid == full_subs)
                def _():
                    pltpu.sync_copy(
                        h_hbm.at[pl.ds(full_subs * rows_per_sub, rem)],
                        buf_v.at[pl.ds(0, rem)])
                    pltpu.sync_copy(
                        buf_v.at[pl.ds(0, rem)],
                        tab_s.at[pl.ds(full_subs * rows_per_sub, rem)])

        pltpu.sync_copy(ei_hbm.at[0, pl.ds(wid * epw, epw)],
                        src_v.at[pl.ds(0, epw)])
        pltpu.sync_copy(ei_hbm.at[1, pl.ds(wid * epw, epw)],
                        dst_v.at[pl.ds(0, epw)])
        pltpu.sync_copy(ew_hbm.at[pl.ds(wid * epw, epw)],
                        ew_v.at[pl.ds(0, epw)])
        zi = jnp.zeros((L,), jnp.int32)
        zf = jnp.zeros((L,), jnp.float32)
        for t in range(tail // L):
            o = epw + t * L
            src_v[o:o + L] = zi
            dst_v[o:o + L] = zi
            ew_v[o:o + L] = zf

        def zero_body(i, carry):
            for k in range(nd):
                buf_v[i, k * L:(k + 1) * L] = zf
            return carry

        lax.fori_loop(0, rows_per_sub, zero_body, 0)
        pltpu.sync_copy(buf_v, acc_s.at[pl.ds(sid * rows_per_sub, rows_per_sub)])
        plsc.subcore_barrier()

        def gather_start(c, b):
            pltpu.async_copy(tab_s.at[src_v.at[pl.ds(c * CHUNK, CHUNK)]],
                             rows_v.at[b], gsem.at[b])

        def gather_wait(c, b):
            pltpu.make_async_copy(
                tab_s.at[src_v.at[pl.ds(c * CHUNK, CHUNK)]],
                rows_v.at[b], gsem.at[b]).wait()

        def scatter_start(c, b):
            pltpu.async_copy(rows_v.at[b],
                             acc_s.at[dst_v.at[pl.ds(c * CHUNK, CHUNK)]],
                             ssem.at[b], add=True)

        def scatter_wait(c, b):
            pltpu.make_async_copy(rows_v.at[b],
                                  acc_s.at[dst_v.at[pl.ds(c * CHUNK, CHUNK)]],
                                  ssem.at[b]).wait()

        def scale(c, b):
            for g in range(CHUNK // L):
                w16 = ew_v[pl.ds(c * CHUNK + g * L, L)]
                for i in range(L):
                    wb = jnp.full((L,), w16[i])
                    row = g * L + i
                    for k in range(nd):
                        rows_v[b, row, k * L:(k + 1) * L] = (
                            rows_v[b, row, k * L:(k + 1) * L] * wb)

        gather_start(0, 0)
        gather_start(1, 1)
        for b in range(nbuf):
            gather_wait(b, b)
            scale(b, b)
            scatter_start(b, b)
            bn = (b + lead) % nbuf
            if b + lead < nbuf:
                gather_start(b + lead, bn)
            else:
                scatter_wait(b + lead - nbuf, bn)
                gather_start(b + lead, bn)

        def ring_cycle(it, carry):
            j = it * nbuf
            for b in range(nbuf):
                c = j + b
                gather_wait(c, b)
                scale(c, b)
                scatter_start(c, b)
                bn = (b + lead) % nbuf

                @pl.when(c + lead < n_chunks)
                def _():
                    scatter_wait(c + lead - nbuf, bn)
                    gather_start(c + lead, bn)
            return carry

        lax.fori_loop(1, n_chunks // nbuf, ring_cycle, 0)
        for b in range(nbuf):
            scatter_wait(n_chunks - nbuf + b, b)

        plsc.subcore_barrier()
        pltpu.sync_copy(acc_s.at[pl.ds(sid * rows_per_sub, rows_per_sub)], buf_v)
        pltpu.sync_copy(buf_v, out_hbm.at[cid, pl.ds(sid * rows_per_sub,
                                                     rows_per_sub)])

    return prop


def kernel(x, edge_index, edge_weight, W1, W2):
    n, _ = x.shape
    hid = W1.shape[1]
    e = edge_index.shape[1]

    if e % (NW * L) != 0:
        pad = NW * L - e % (NW * L)
        edge_index = jnp.concatenate(
            [edge_index, jnp.zeros((2, pad), edge_index.dtype)], axis=1)
        edge_weight = jnp.concatenate(
            [edge_weight, jnp.zeros((pad,), edge_weight.dtype)])
        e += pad

    n_acc = -(-n // (NS * 8)) * (NS * 8)

    prop1 = _make_propagate(n_acc, hid, n, e, False)
    prop2 = _make_propagate(n_acc, hid, n_acc, e, True)

    h0 = _matmul_tc(x, W1)
    p1 = prop1(h0, edge_index, edge_weight)
    p2 = prop2(p1, edge_index, edge_weight)
    return _add_matmul_tc(p2[0, :n], p2[1, :n], W2)


# speedup vs baseline: 24.7970x; 1.0499x over previous
import functools

import jax
import jax.numpy as jnp
from jax import lax
from jax.experimental import pallas as pl
from jax.experimental.pallas import tpu as pltpu
from jax.experimental.pallas import tpu_sc as plsc

NC = 2
NS = 16
L = 16
NW = NC * NS
CHUNK = 128


def _mm_body(x_ref, w_ref, o_ref):
    o_ref[...] = jnp.dot(x_ref[...], w_ref[...],
                         preferred_element_type=jnp.float32,
                         precision=jax.lax.Precision.HIGHEST)


def _matmul_tc(x, w, blk=2000):
    n, k = x.shape
    m = w.shape[1]
    return pl.pallas_call(
        _mm_body,
        grid=(n // blk,),
        in_specs=[pl.BlockSpec((blk, k), lambda i: (i, 0)),
                  pl.BlockSpec((k, m), lambda i: (0, 0))],
        out_specs=pl.BlockSpec((blk, m), lambda i: (i, 0)),
        out_shape=jax.ShapeDtypeStruct((n, m), jnp.float32),
    )(x, w)


def _relu_add_body(a_ref, b_ref, o_ref):
    o_ref[...] = jnp.maximum(a_ref[...] + b_ref[...], 0.0)


def _relu_add_tc(a, b, blk=2000):
    n, m = a.shape
    return pl.pallas_call(
        _relu_add_body,
        grid=(n // blk,),
        in_specs=[pl.BlockSpec((blk, m), lambda i: (i, 0)),
                  pl.BlockSpec((blk, m), lambda i: (i, 0))],
        out_specs=pl.BlockSpec((blk, m), lambda i: (i, 0)),
        out_shape=jax.ShapeDtypeStruct((n, m), jnp.float32),
    )(a, b)


def _add_matmul_body(a_ref, b_ref, w_ref, o_ref):
    o_ref[...] = jnp.dot(a_ref[0] + b_ref[0], w_ref[...],
                         preferred_element_type=jnp.float32,
                         precision=jax.lax.Precision.HIGHEST)


def _add_matmul_tc(p, w, n, blk=2000):
    k = p.shape[2]
    m = w.shape[1]
    return pl.pallas_call(
        _add_matmul_body,
        grid=(n // blk,),
        in_specs=[pl.BlockSpec((1, blk, k), lambda i: (0, i, 0)),
                  pl.BlockSpec((1, blk, k), lambda i: (1, i, 0)),
                  pl.BlockSpec((k, m), lambda i: (0, 0))],
        out_specs=pl.BlockSpec((blk, m), lambda i: (i, 0)),
        out_shape=jax.ShapeDtypeStruct((n, m), jnp.float32),
    )(p, p, w)


@functools.cache
def _make_propagate(n_acc, d, n_in, e, staged):
    epw = e // NW
    assert e % (NW * L) == 0
    n_chunks = -(-epw // CHUNK)
    n_chunks = max(-(-n_chunks // 4) * 4, 8)
    epw_pad = n_chunks * CHUNK
    tail = epw_pad - epw
    assert tail % L == 0
    rows_per_sub = n_acc // NS
    nd = d // L
    mesh = plsc.VectorSubcoreMesh(core_axis_name="c", subcore_axis_name="s")

    nbuf = 4
    lead = 2

    @functools.partial(
        pl.kernel,
        out_type=jax.ShapeDtypeStruct((NC, n_acc, d), jnp.float32),
        mesh=mesh,
        scratch_types=[
            pltpu.VMEM((epw_pad,), jnp.int32),
            pltpu.VMEM((epw_pad,), jnp.int32),
            pltpu.VMEM((epw_pad,), jnp.float32),
            pltpu.VMEM((nbuf, CHUNK, d), jnp.float32),
            pltpu.VMEM((rows_per_sub, d), jnp.float32),
            pltpu.VMEM((rows_per_sub, d), jnp.float32),
            pltpu.VMEM_SHARED((n_acc, d), jnp.float32),
            pltpu.VMEM_SHARED((n_acc, d), jnp.float32),
            pltpu.SemaphoreType.DMA((nbuf,)),
            pltpu.SemaphoreType.DMA((nbuf,)),
        ],
        compiler_params=pltpu.CompilerParams(use_tc_tiling_on_sc=False),
    )
    def prop(h_hbm, ei_hbm, ew_hbm, out_hbm,
             src_v, dst_v, ew_v, rows_v, buf_v, buf2_v, acc_s, tab_s,
             gsem, ssem):
        cid = lax.axis_index("c")
        sid = lax.axis_index("s")
        wid = cid * NS + sid

        if staged:
            pltpu.sync_copy(h_hbm.at[0, pl.ds(sid * rows_per_sub,
                                              rows_per_sub)], buf_v)
            pltpu.sync_copy(h_hbm.at[1, pl.ds(sid * rows_per_sub,
                                              rows_per_sub)], buf2_v)

            def stage_body(i, carry):
                for k in range(nd):
                    a = buf_v[i, k * L:(k + 1) * L]
                    b = buf2_v[i, k * L:(k + 1) * L]
                    buf_v[i, k * L:(k + 1) * L] = jnp.maximum(a + b, 0.0)
                return carry

            lax.fori_loop(0, rows_per_sub, stage_body, 0)
            pltpu.sync_copy(buf_v,
                            tab_s.at[pl.ds(sid * rows_per_sub, rows_per_sub)])
        else:
            n_rows = h_hbm.shape[0]
            full_subs = n_rows // rows_per_sub
            rem = n_rows % rows_per_sub

            @pl.when(sid < full_subs)
            def _():
                pltpu.sync_copy(
                    h_hbm.at[pl.ds(sid * rows_per_sub, rows_per_sub)], buf_v)
                pltpu.sync_copy(
                    buf_v, tab_s.at[pl.ds(sid * rows_per_sub, rows_per_sub)])

            if rem > 0:
                @pl.when(sid == full_subs)
                def _():
                    pltpu.sync_copy(
                        h_hbm.at[pl.ds(full_subs * rows_per_sub, rem)],
                        buf_v.at[pl.ds(0, rem)])
                    pltpu.sync_copy(
                        buf_v.at[pl.ds(0, rem)],
                        tab_s.at[pl.ds(full_subs * rows_per_sub, rem)])

        pltpu.sync_copy(ei_hbm.at[0, pl.ds(wid * epw, epw)],
                        src_v.at[pl.ds(0, epw)])
        pltpu.sync_copy(ei_hbm.at[1, pl.ds(wid * epw, epw)],
                        dst_v.at[pl.ds(0, epw)])
        pltpu.sync_copy(ew_hbm.at[pl.ds(wid * epw, epw)],
                        ew_v.at[pl.ds(0, epw)])
        zi = jnp.zeros((L,), jnp.int32)
        zf = jnp.zeros((L,), jnp.float32)
        for t in range(tail // L):
            o = epw + t * L
            src_v[o:o + L] = zi
            dst_v[o:o + L] = zi
            ew_v[o:o + L] = zf

        def zero_body(i, carry):
            for k in range(nd):
                buf_v[i, k * L:(k + 1) * L] = zf
            return carry

        lax.fori_loop(0, rows_per_sub, zero_body, 0)
        pltpu.sync_copy(buf_v, acc_s.at[pl.ds(sid * rows_per_sub, rows_per_sub)])
        plsc.subcore_barrier()

        def gather_start(c, b):
            pltpu.async_copy(tab_s.at[src_v.at[pl.ds(c * CHUNK, CHUNK)]],
                             rows_v.at[b], gsem.at[b])

        def gather_wait(c, b):
            pltpu.make_async_copy(
                tab_s.at[src_v.at[pl.ds(c * CHUNK, CHUNK)]],
                rows_v.at[b], gsem.at[b]).wait()

        def scatter_start(c, b):
            pltpu.async_copy(rows_v.at[b],
                             acc_s.at[dst_v.at[pl.ds(c * CHUNK, CHUNK)]],
                             ssem.at[b], add=True)

        def scatter_wait(c, b):
            pltpu.make_async_copy(rows_v.at[b],
                                  acc_s.at[dst_v.at[pl.ds(c * CHUNK, CHUNK)]],
                                  ssem.at[b]).wait()

        def scale(c, b):
            for g in range(CHUNK // L):
                w16 = ew_v[pl.ds(c * CHUNK + g * L, L)]
                for i in range(L):
                    wb = jnp.full((L,), w16[i])
                    row = g * L + i
                    for k in range(nd):
                        rows_v[b, row, k * L:(k + 1) * L] = (
                            rows_v[b, row, k * L:(k + 1) * L] * wb)

        gather_start(0, 0)
        gather_start(1, 1)
        for b in range(nbuf):
            gather_wait(b, b)
            scale(b, b)
            scatter_start(b, b)
            bn = (b + lead) % nbuf
            if b + lead < nbuf:
                gather_start(b + lead, bn)
            else:
                scatter_wait(b + lead - nbuf, bn)
                gather_start(b + lead, bn)

        def ring_cycle(it, carry):
            j = it * nbuf
            for b in range(nbuf):
                c = j + b
                gather_wait(c, b)
                scale(c, b)
                scatter_start(c, b)
                bn = (b + lead) % nbuf

                @pl.when(c + lead < n_chunks)
                def _():
                    scatter_wait(c + lead - nbuf, bn)
                    gather_start(c + lead, bn)
            return carry

        lax.fori_loop(1, n_chunks // nbuf, ring_cycle, 0)
        for b in range(nbuf):
            scatter_wait(n_chunks - nbuf + b, b)

        plsc.subcore_barrier()
        pltpu.sync_copy(acc_s.at[pl.ds(sid * rows_per_sub, rows_per_sub)], buf_v)
        pltpu.sync_copy(buf_v, out_hbm.at[cid, pl.ds(sid * rows_per_sub,
                                                     rows_per_sub)])

    return prop


def kernel(x, edge_index, edge_weight, W1, W2):
    n, _ = x.shape
    hid = W1.shape[1]
    e = edge_index.shape[1]

    if e % (NW * L) != 0:
        pad = NW * L - e % (NW * L)
        edge_index = jnp.concatenate(
            [edge_index, jnp.zeros((2, pad), edge_index.dtype)], axis=1)
        edge_weight = jnp.concatenate(
            [edge_weight, jnp.zeros((pad,), edge_weight.dtype)])
        e += pad

    n_acc = -(-n // (NS * 8)) * (NS * 8)

    prop1 = _make_propagate(n_acc, hid, n, e, False)
    prop2 = _make_propagate(n_acc, hid, n_acc, e, True)

    h0 = _matmul_tc(x, W1)
    p1 = prop1(h0, edge_index, edge_weight)
    p2 = prop2(p1, edge_index, edge_weight)
    return _add_matmul_tc(p2, W2, n)
